# Initial kernel scaffold; baseline (speedup 1.0000x reference)
import jax
import jax.numpy as jnp
from jax.experimental import pallas as pl


def kernel(address, tourist_type, price_sensitive, like_type, targets, attention, addr_table, tt_table, ps_table, lt_table, tg_table, at_table, W1, b1, W2, b2):
    raise NotImplementedError("write your pallas kernel here")


# speedup vs baseline: 8.4940x; 8.4940x over previous
import functools

import jax
import jax.numpy as jnp
from jax import lax
from jax.experimental import pallas as pl
from jax.experimental.pallas import tpu as pltpu
from jax.experimental.pallas import tpu_sc as plsc

B = 16384
L = 20
ED = 32
HID = 128
OUT = 64

NC = 2
NS = 16
NW = NC * NS
BPW = B // NW
ACH = 128
NACH = BPW // ACH

_MESH = plsc.VectorSubcoreMesh(core_axis_name="c", subcore_axis_name="s")


def _embed_body(addr_idx2, tt_idx, ps_idx, ltT, tgT, atT,
                addr_tab, tt_flat, ps_flat, lt_flat, tg_flat, at_flat,
                addr_out, tt_out, ps_out, lt_out, tg_out, at_out,
                iv2, rows_v, tabf, idxf, outf, sem_a, sem_i):
    wid = lax.axis_index("s") * NC + lax.axis_index("c")
    base = wid * BPW

    pltpu.sync_copy(addr_idx2.at[pl.ds(wid * NACH, NACH)], iv2)
    addr_descs = [
        pltpu.async_copy(addr_tab.at[iv2.at[j]],
                         rows_v.at[pl.ds(j * ACH, ACH)], sem_a)
        for j in range(NACH)
    ]

    iota16 = lax.iota(jnp.int32, 16)

    def pool(tab_hbm, nwords, idx_hbm, lf, out_hbm, scale):
        pltpu.sync_copy(tab_hbm, tabf.at[pl.ds(0, nwords)])
        if lf == 1:
            pltpu.sync_copy(idx_hbm.at[pl.ds(base, BPW)],
                            idxf.at[pl.ds(0, BPW)])
        else:
            descs = [
                pltpu.async_copy(idx_hbm.at[li, pl.ds(base, BPW)],
                                 idxf.at[pl.ds(li * BPW, BPW)], sem_i)
                for li in range(lf)
            ]
            for d in descs:
                d.wait()

        def gbody(g, carry):
            g16 = g * 16

            def lbody(l, accs):
                iv = idxf[pl.ds(l * BPW + g16, 16)]
                a0 = iv * ED
                return tuple(
                    accs[d] + plsc.load_gather(tabf, [a0 + d])
                    for d in range(ED)
                )

            accs = tuple(jnp.zeros((16,), jnp.float32) for _ in range(ED))
            accs = lax.fori_loop(0, lf, lbody, accs)
            rowv = (g16 + iota16) * ED
            for d in range(ED):
                v = accs[d] * scale if scale != 1.0 else accs[d]
                plsc.store_scatter(outf, [rowv + d], v)
            return carry

        lax.fori_loop(0, BPW // 16, gbody, 0)
        pltpu.sync_copy(outf, out_hbm.at[pl.ds(base * ED, BPW * ED)])

    pool(tt_flat, 7 * ED, tt_idx, 1, tt_out, 1.0)
    pool(ps_flat, 2 * ED, ps_idx, 1, ps_out, 1.0)
    pool(lt_flat, 1000 * ED, ltT, L, lt_out, 1.0 / L)
    pool(tg_flat, 1000 * ED, tgT, L, tg_out, 1.0 / L)
    pool(at_flat, 1000 * ED, atT, L, at_out, 1.0 / L)

    for d in addr_descs:
        d.wait()
    pltpu.sync_copy(rows_v, addr_out.at[pl.ds(base, BPW)])


@jax.jit
def _embed(address, tourist_type, price_sensitive, ltT, tgT, atT,
           addr_table, tt_table, ps_table, lt_table, tg_table, at_table):
    f32 = jnp.float32
    out_type = [
        jax.ShapeDtypeStruct((B, ED), f32),
        jax.ShapeDtypeStruct((B * ED,), f32),
        jax.ShapeDtypeStruct((B * ED,), f32),
        jax.ShapeDtypeStruct((B * ED,), f32),
        jax.ShapeDtypeStruct((B * ED,), f32),
        jax.ShapeDtypeStruct((B * ED,), f32),
    ]
    scratch = [
        pltpu.VMEM((NACH, ACH), jnp.int32),
        pltpu.VMEM((BPW, ED), f32),
        pltpu.VMEM((1000 * ED,), f32),
        pltpu.VMEM((L * BPW,), jnp.int32),
        pltpu.VMEM((BPW * ED,), f32),
        pltpu.SemaphoreType.DMA,
        pltpu.SemaphoreType.DMA,
    ]
    fn = pl.kernel(_embed_body, out_type=out_type, mesh=_MESH,
                   scratch_types=scratch,
                   compiler_params=pltpu.CompilerParams(
                       needs_layout_passes=False,
                       use_tc_tiling_on_sc=False))
    return fn(address.reshape(B // ACH, ACH), tourist_type, price_sensitive,
              ltT, tgT, atT, addr_table,
              tt_table.reshape(-1), ps_table.reshape(-1),
              lt_table.reshape(-1), tg_table.reshape(-1),
              at_table.reshape(-1))


def _mlp_body(a, t, p, lt, tg, at, w1, b1, w2, b2, o):
    x = jnp.concatenate(
        [a[...], t[...], p[...], lt[...], tg[...], at[...]], axis=1)
    h = jnp.dot(x, w1[...], preferred_element_type=jnp.float32) + b1[...]
    h = jnp.maximum(h, 0.0)
    y = jnp.dot(h, w2[...], preferred_element_type=jnp.float32) + b2[...]
    ss = jnp.sum(y * y, axis=1, keepdims=True)
    n = jnp.maximum(jnp.sqrt(ss), 1e-12)
    o[...] = y / n


@functools.partial(jax.jit, static_argnames=("bm",))
def _mlp(a, t, p, lt, tg, at, W1, b1, W2, b2, bm=2048):
    grid = (B // bm,)
    eb = lambda i: (i, 0)
    z = lambda i: (0, 0)
    return pl.pallas_call(
        _mlp_body,
        grid=grid,
        in_specs=[
            pl.BlockSpec((bm, ED), eb), pl.BlockSpec((bm, ED), eb),
            pl.BlockSpec((bm, ED), eb), pl.BlockSpec((bm, ED), eb),
            pl.BlockSpec((bm, ED), eb), pl.BlockSpec((bm, ED), eb),
            pl.BlockSpec((6 * ED, HID), z), pl.BlockSpec((1, HID), z),
            pl.BlockSpec((HID, OUT), z), pl.BlockSpec((1, OUT), z),
        ],
        out_specs=pl.BlockSpec((bm, OUT), eb),
        out_shape=jax.ShapeDtypeStruct((B, OUT), jnp.float32),
    )(a, t, p, lt, tg, at, W1, b1, W2, b2)


def kernel(address, tourist_type, price_sensitive, like_type, targets,
           attention, addr_table, tt_table, ps_table, lt_table, tg_table,
           at_table, W1, b1, W2, b2):
    i32 = jnp.int32
    address = address.astype(i32)
    tourist_type = tourist_type.astype(i32)
    price_sensitive = price_sensitive.astype(i32)
    ltT = like_type.astype(i32).T
    tgT = targets.astype(i32).T
    atT = attention.astype(i32).T

    a, t, p, lt, tg, at = _embed(
        address, tourist_type, price_sensitive, ltT, tgT, atT,
        addr_table, tt_table, ps_table, lt_table, tg_table, at_table)
    t = t.reshape(B, ED)
    p = p.reshape(B, ED)
    lt = lt.reshape(B, ED)
    tg = tg.reshape(B, ED)
    at = at.reshape(B, ED)
    return _mlp(a, t, p, lt, tg, at, W1, b1.reshape(1, HID), W2,
                b2.reshape(1, OUT))


# speedup vs baseline: 20.6145x; 2.4269x over previous
import functools

import jax
import jax.numpy as jnp
from jax import lax
from jax.experimental import pallas as pl
from jax.experimental.pallas import tpu as pltpu
from jax.experimental.pallas import tpu_sc as plsc

B = 16384
L = 20
ED = 32
HID = 128
OUT = 64

NC = 2
NS = 16
NW = NC * NS
BPW = B // NW
ACH = 128
NACH = BPW // ACH

_MESH = plsc.VectorSubcoreMesh(core_axis_name="c", subcore_axis_name="s")


def _embed_body(addr_idx2, tt_idx, ps_idx, ltT, tgT, atT,
                addr_tab, tt_flat, ps_flat, lt_flat, tg_flat, at_flat,
                addr_out, tt_out, ps_out, lt_out, tg_out, at_out,
                iv2, rows_v, tabf, idxf, outf, sem_a, sem_i):
    wid = lax.axis_index("s") * NC + lax.axis_index("c")
    base = wid * BPW

    pltpu.sync_copy(addr_idx2.at[pl.ds(wid * NACH, NACH)], iv2)
    addr_descs = [
        pltpu.async_copy(addr_tab.at[iv2.at[j]],
                         rows_v.at[pl.ds(j * ACH, ACH)], sem_a)
        for j in range(NACH)
    ]

    iota16 = lax.iota(jnp.int32, 16)

    def pool(tab_hbm, nrows, idx_hbm, lf, out_hbm, scale):
        pltpu.sync_copy(tab_hbm, tabf.at[pl.ds(0, nrows * ED)])
        if lf == 1:
            pltpu.sync_copy(idx_hbm.at[pl.ds(base, BPW)],
                            idxf.at[pl.ds(0, BPW)])
        else:
            descs = [
                pltpu.async_copy(idx_hbm.at[li, pl.ds(base, BPW)],
                                 idxf.at[pl.ds(li * BPW, BPW)], sem_i)
                for li in range(lf)
            ]
            for d in descs:
                d.wait()

        def gbody(g, carry):
            g16 = g * 16

            def lbody(l, accs):
                iv = idxf[pl.ds(l * BPW + g16, 16)]
                return tuple(
                    accs[d] + plsc.load_gather(tabf, [iv + d * nrows])
                    for d in range(ED)
                )

            accs = tuple(jnp.zeros((16,), jnp.float32) for _ in range(ED))
            accs = lax.fori_loop(0, lf, lbody, accs)
            rowv = g16 + iota16
            for d in range(ED):
                v = accs[d] * scale if scale != 1.0 else accs[d]
                plsc.store_scatter(outf, [rowv, jnp.full((16,), d, jnp.int32)], v)
            return carry

        lax.fori_loop(0, BPW // 16, gbody, 0)
        pltpu.sync_copy(outf.at[:, pl.ds(0, ED)],
                        out_hbm.at[pl.ds(base, BPW)])

    pool(tt_flat, 7, tt_idx, 1, tt_out, 1.0)
    pool(ps_flat, 2, ps_idx, 1, ps_out, 1.0)
    pool(lt_flat, 1000, ltT, L, lt_out, 1.0 / L)
    pool(tg_flat, 1000, tgT, L, tg_out, 1.0 / L)
    pool(at_flat, 1000, atT, L, at_out, 1.0 / L)

    for d in addr_descs:
        d.wait()
    pltpu.sync_copy(rows_v, addr_out.at[pl.ds(base, BPW)])


@jax.jit
def _embed(address, tourist_type, price_sensitive, ltT, tgT, atT,
           addr_table, tt_table, ps_table, lt_table, tg_table, at_table):
    f32 = jnp.float32
    out_type = [
        jax.ShapeDtypeStruct((B, ED), f32),
        jax.ShapeDtypeStruct((B, ED), f32),
        jax.ShapeDtypeStruct((B, ED), f32),
        jax.ShapeDtypeStruct((B, ED), f32),
        jax.ShapeDtypeStruct((B, ED), f32),
        jax.ShapeDtypeStruct((B, ED), f32),
    ]
    scratch = [
        pltpu.VMEM((NACH, ACH), jnp.int32),
        pltpu.VMEM((BPW, ED), f32),
        pltpu.VMEM((1000 * ED,), f32),
        pltpu.VMEM((L * BPW,), jnp.int32),
        pltpu.VMEM((BPW, ED + 1), f32),
        pltpu.SemaphoreType.DMA,
        pltpu.SemaphoreType.DMA,
    ]
    fn = pl.kernel(_embed_body, out_type=out_type, mesh=_MESH,
                   scratch_types=scratch,
                   compiler_params=pltpu.CompilerParams(
                       needs_layout_passes=False,
                       use_tc_tiling_on_sc=False))
    return fn(address.reshape(B // ACH, ACH), tourist_type, price_sensitive,
              ltT, tgT, atT, addr_table,
              tt_table.T.reshape(-1), ps_table.T.reshape(-1),
              lt_table.T.reshape(-1), tg_table.T.reshape(-1),
              at_table.T.reshape(-1))


def _mlp_body(a, t, p, lt, tg, at, w1, b1, w2, b2, o):
    x = jnp.concatenate(
        [a[...], t[...], p[...], lt[...], tg[...], at[...]], axis=1)
    h = jnp.dot(x, w1[...], preferred_element_type=jnp.float32) + b1[...]
    h = jnp.maximum(h, 0.0)
    y = jnp.dot(h, w2[...], preferred_element_type=jnp.float32) + b2[...]
    ss = jnp.sum(y * y, axis=1, keepdims=True)
    n = jnp.maximum(jnp.sqrt(ss), 1e-12)
    o[...] = y / n


@functools.partial(jax.jit, static_argnames=("bm",))
def _mlp(a, t, p, lt, tg, at, W1, b1, W2, b2, bm=2048):
    grid = (B // bm,)
    eb = lambda i: (i, 0)
    z = lambda i: (0, 0)
    return pl.pallas_call(
        _mlp_body,
        grid=grid,
        in_specs=[
            pl.BlockSpec((bm, ED), eb), pl.BlockSpec((bm, ED), eb),
            pl.BlockSpec((bm, ED), eb), pl.BlockSpec((bm, ED), eb),
            pl.BlockSpec((bm, ED), eb), pl.BlockSpec((bm, ED), eb),
            pl.BlockSpec((6 * ED, HID), z), pl.BlockSpec((1, HID), z),
            pl.BlockSpec((HID, OUT), z), pl.BlockSpec((1, OUT), z),
        ],
        out_specs=pl.BlockSpec((bm, OUT), eb),
        out_shape=jax.ShapeDtypeStruct((B, OUT), jnp.float32),
    )(a, t, p, lt, tg, at, W1, b1, W2, b2)


def kernel(address, tourist_type, price_sensitive, like_type, targets,
           attention, addr_table, tt_table, ps_table, lt_table, tg_table,
           at_table, W1, b1, W2, b2):
    i32 = jnp.int32
    address = address.astype(i32)
    tourist_type = tourist_type.astype(i32)
    price_sensitive = price_sensitive.astype(i32)
    ltT = like_type.astype(i32).T
    tgT = targets.astype(i32).T
    atT = attention.astype(i32).T

    a, t, p, lt, tg, at = _embed(
        address, tourist_type, price_sensitive, ltT, tgT, atT,
        addr_table, tt_table, ps_table, lt_table, tg_table, at_table)
    return _mlp(a, t, p, lt, tg, at, W1, b1.reshape(1, HID), W2,
                b2.reshape(1, OUT))


# speedup vs baseline: 25.4136x; 1.2328x over previous
import functools

import jax
import jax.numpy as jnp
from jax import lax
from jax.experimental import pallas as pl
from jax.experimental.pallas import tpu as pltpu
from jax.experimental.pallas import tpu_sc as plsc

B = 16384
L = 20
ED = 32
HID = 128
OUT = 64

NC = 2
NS = 16
NW = NC * NS
BPW = B // NW
ACH = 128
NACH = BPW // ACH

_MESH = plsc.VectorSubcoreMesh(core_axis_name="c", subcore_axis_name="s")


def _embed_body(addr_idx2, tt_idx, ps_idx, ltT, tgT, atT,
                addr_tab, tt_flat, ps_flat, lt_flat, tg_flat, at_flat,
                addr_out, tt_out, ps_out, lt_out, tg_out, at_out,
                iv2, rows_v, tabf, tabp, idxf, outf, sem_a, sem_i):
    wid = lax.axis_index("s") * NC + lax.axis_index("c")
    base = wid * BPW

    pltpu.sync_copy(addr_idx2.at[pl.ds(wid * NACH, NACH)], iv2)
    addr_descs = [
        pltpu.async_copy(addr_tab.at[iv2.at[j]],
                         rows_v.at[pl.ds(j * ACH, ACH)], sem_a)
        for j in range(NACH)
    ]

    iota16 = lax.iota(jnp.int32, 16)

    def pool(tab_hbm, nrows, idx_hbm, lf, out_hbm, scale, packed):
        if packed:
            pltpu.sync_copy(tab_hbm, tabp)
        else:
            pltpu.sync_copy(tab_hbm, tabf.at[pl.ds(0, nrows * ED)])
        if lf == 1:
            pltpu.sync_copy(idx_hbm.at[pl.ds(base, BPW)],
                            idxf.at[pl.ds(0, BPW)])
        else:
            descs = [
                pltpu.async_copy(idx_hbm.at[li, pl.ds(base, BPW)],
                                 idxf.at[pl.ds(li * BPW, BPW)], sem_i)
                for li in range(lf)
            ]
            for d in descs:
                d.wait()

        def gbody(g, carry):
            g16 = g * 16

            def lbody(l, accs):
                iv = idxf[pl.ds(l * BPW + g16, 16)]
                if not packed:
                    return tuple(
                        accs[d] + plsc.load_gather(tabf, [iv + d * nrows])
                        for d in range(ED)
                    )
                new = list(accs)
                for dp in range(ED // 2):
                    w = plsc.load_gather(tabp, [iv + dp * nrows])
                    lo = plsc.bitcast(w << 16, jnp.float32)
                    hi = plsc.bitcast(w & jnp.int32(-65536), jnp.float32)
                    new[2 * dp] = new[2 * dp] + lo
                    new[2 * dp + 1] = new[2 * dp + 1] + hi
                return tuple(new)

            accs = tuple(jnp.zeros((16,), jnp.float32) for _ in range(ED))
            accs = lax.fori_loop(0, lf, lbody, accs)
            rowv = g16 + iota16
            for d in range(ED):
                v = accs[d] * scale if scale != 1.0 else accs[d]
                plsc.store_scatter(outf, [rowv, jnp.full((16,), d, jnp.int32)], v)
            return carry

        lax.fori_loop(0, BPW // 16, gbody, 0)
        pltpu.sync_copy(outf.at[:, pl.ds(0, ED)],
                        out_hbm.at[pl.ds(base, BPW)])

    pool(tt_flat, 7, tt_idx, 1, tt_out, 1.0, False)
    pool(ps_flat, 2, ps_idx, 1, ps_out, 1.0, False)
    pool(lt_flat, 1000, ltT, L, lt_out, 1.0 / L, True)
    pool(tg_flat, 1000, tgT, L, tg_out, 1.0 / L, True)
    pool(at_flat, 1000, atT, L, at_out, 1.0 / L, True)

    for d in addr_descs:
        d.wait()
    pltpu.sync_copy(rows_v, addr_out.at[pl.ds(base, BPW)])


def _pack_bf16_T(tab):
    t = tab.astype(jnp.bfloat16).reshape(-1, ED // 2, 2)
    w = jax.lax.bitcast_convert_type(t, jnp.int32)
    return w.T.reshape(-1)


@jax.jit
def _embed(address, tourist_type, price_sensitive, ltT, tgT, atT,
           addr_table, tt_table, ps_table, lt_table, tg_table, at_table):
    f32 = jnp.float32
    out_type = [
        jax.ShapeDtypeStruct((B, ED), f32),
        jax.ShapeDtypeStruct((B, ED), f32),
        jax.ShapeDtypeStruct((B, ED), f32),
        jax.ShapeDtypeStruct((B, ED), f32),
        jax.ShapeDtypeStruct((B, ED), f32),
        jax.ShapeDtypeStruct((B, ED), f32),
    ]
    scratch = [
        pltpu.VMEM((NACH, ACH), jnp.int32),
        pltpu.VMEM((BPW, ED), f32),
        pltpu.VMEM((7 * ED,), f32),
        pltpu.VMEM((1000 * ED // 2,), jnp.int32),
        pltpu.VMEM((L * BPW,), jnp.int32),
        pltpu.VMEM((BPW, ED + 1), f32),
        pltpu.SemaphoreType.DMA,
        pltpu.SemaphoreType.DMA,
    ]
    fn = pl.kernel(_embed_body, out_type=out_type, mesh=_MESH,
                   scratch_types=scratch,
                   compiler_params=pltpu.CompilerParams(
                       needs_layout_passes=False,
                       use_tc_tiling_on_sc=False))
    return fn(address.reshape(B // ACH, ACH), tourist_type, price_sensitive,
              ltT, tgT, atT, addr_table,
              tt_table.T.reshape(-1), ps_table.T.reshape(-1),
              _pack_bf16_T(lt_table), _pack_bf16_T(tg_table),
              _pack_bf16_T(at_table))


def _mlp_body(a, t, p, lt, tg, at, w1, b1, w2, b2, o):
    x = jnp.concatenate(
        [a[...], t[...], p[...], lt[...], tg[...], at[...]], axis=1)
    h = jnp.dot(x, w1[...], preferred_element_type=jnp.float32) + b1[...]
    h = jnp.maximum(h, 0.0)
    y = jnp.dot(h, w2[...], preferred_element_type=jnp.float32) + b2[...]
    ss = jnp.sum(y * y, axis=1, keepdims=True)
    n = jnp.maximum(jnp.sqrt(ss), 1e-12)
    o[...] = y / n


@functools.partial(jax.jit, static_argnames=("bm",))
def _mlp(a, t, p, lt, tg, at, W1, b1, W2, b2, bm=2048):
    grid = (B // bm,)
    eb = lambda i: (i, 0)
    z = lambda i: (0, 0)
    return pl.pallas_call(
        _mlp_body,
        grid=grid,
        in_specs=[
            pl.BlockSpec((bm, ED), eb), pl.BlockSpec((bm, ED), eb),
            pl.BlockSpec((bm, ED), eb), pl.BlockSpec((bm, ED), eb),
            pl.BlockSpec((bm, ED), eb), pl.BlockSpec((bm, ED), eb),
            pl.BlockSpec((6 * ED, HID), z), pl.BlockSpec((1, HID), z),
            pl.BlockSpec((HID, OUT), z), pl.BlockSpec((1, OUT), z),
        ],
        out_specs=pl.BlockSpec((bm, OUT), eb),
        out_shape=jax.ShapeDtypeStruct((B, OUT), jnp.float32),
    )(a, t, p, lt, tg, at, W1, b1, W2, b2)


def kernel(address, tourist_type, price_sensitive, like_type, targets,
           attention, addr_table, tt_table, ps_table, lt_table, tg_table,
           at_table, W1, b1, W2, b2):
    i32 = jnp.int32
    address = address.astype(i32)
    tourist_type = tourist_type.astype(i32)
    price_sensitive = price_sensitive.astype(i32)
    ltT = like_type.astype(i32).T
    tgT = targets.astype(i32).T
    atT = attention.astype(i32).T

    a, t, p, lt, tg, at = _embed(
        address, tourist_type, price_sensitive, ltT, tgT, atT,
        addr_table, tt_table, ps_table, lt_table, tg_table, at_table)
    return _mlp(a, t, p, lt, tg, at, W1, b1.reshape(1, HID), W2,
                b2.reshape(1, OUT))
